# Initial kernel scaffold; baseline (speedup 1.0000x reference)
import jax
import jax.numpy as jnp
from jax.experimental import pallas as pl


def kernel(level_table, type_table, feature_table, exchange_table, pair_table, W, b, level_indices, type_indices, feature_indices, exchange_indices, pair_indices):
    raise NotImplementedError("write your pallas kernel here")


# speedup vs baseline: 1.1263x; 1.1263x over previous
import functools

import jax
import jax.numpy as jnp
from jax import lax
from jax.experimental import pallas as pl
from jax.experimental.pallas import tpu as pltpu
from jax.experimental.pallas import tpu_sc as plsc

N = 100000
D = 128
CHUNK = 80
NCHUNKS = N // CHUNK
LANES = 16


def _project_body(lt, tt, ft, et, pt, w1, w2, w3, w4, w5, b,
                  p1, p2, p3, p4, p5):
    dn = (((1,), (1,)), ((), ()))
    p1[...] = lax.dot_general(lt[...], w1[...], dn,
                              preferred_element_type=jnp.float32) + b[...]
    p2[...] = lax.dot_general(tt[...], w2[...], dn,
                              preferred_element_type=jnp.float32)
    p3[...] = lax.dot_general(ft[...], w3[...], dn,
                              preferred_element_type=jnp.float32)
    p4[...] = lax.dot_general(et[...], w4[...], dn,
                              preferred_element_type=jnp.float32)
    p5[...] = lax.dot_general(pt[...], w5[...], dn,
                              preferred_element_type=jnp.float32)


def _project_tables(lt, tt, ft, et, pt, W, b):
    w1 = W[:, 0:25]
    w2 = W[:, 25:50]
    w3 = W[:, 50:75]
    w4 = W[:, 75:100]
    w5 = W[:, 100:128]
    b2 = b.reshape(1, D)
    shapes = [jax.ShapeDtypeStruct((t.shape[0], D), jnp.float32)
              for t in (lt, tt, ft, et, pt)]
    return pl.pallas_call(
        _project_body,
        out_shape=shapes,
    )(lt, tt, ft, et, pt, w1, w2, w3, w4, w5, b2)


def _gather_sum_body(p1, p2, p3, p4, p5, li, ti, fi, ei, pi, out,
                     i1, i2, i3, i4, i5, b1, b2, b3, b4, b5, sem):
    info = plsc.get_sparse_core_info()
    nc = info.num_cores
    nw = nc * info.num_subcores
    wid = lax.axis_index("s") * nc + lax.axis_index("c")
    max_iters = (NCHUNKS + nw - 1) // nw

    def chunk_body(it, _):
        c = wid + it * nw

        @pl.when(c < NCHUNKS)
        def _():
            base = c * CHUNK
            pltpu.sync_copy(li.at[pl.ds(base, CHUNK)], i1)
            pltpu.sync_copy(ti.at[pl.ds(base, CHUNK)], i2)
            pltpu.sync_copy(fi.at[pl.ds(base, CHUNK)], i3)
            pltpu.sync_copy(ei.at[pl.ds(base, CHUNK)], i4)
            pltpu.sync_copy(pi.at[pl.ds(base, CHUNK)], i5)
            cps = [
                pltpu.async_copy(p1.at[i1], b1, sem),
                pltpu.async_copy(p2.at[i2], b2, sem),
                pltpu.async_copy(p3.at[i3], b3, sem),
                pltpu.async_copy(p4.at[i4], b4, sem),
                pltpu.async_copy(p5.at[i5], b5, sem),
            ]
            for cp in cps:
                cp.wait()

            def row_body(r, _):
                for g in range(D // LANES):
                    col = g * LANES
                    s = (b1[r, pl.ds(col, LANES)] + b2[r, pl.ds(col, LANES)]
                         + b3[r, pl.ds(col, LANES)] + b4[r, pl.ds(col, LANES)]
                         + b5[r, pl.ds(col, LANES)])
                    b1[r, pl.ds(col, LANES)] = s
                return 0

            lax.fori_loop(0, CHUNK, row_body, 0)
            pltpu.sync_copy(b1, out.at[pl.ds(base, CHUNK)])
        return 0

    lax.fori_loop(0, max_iters, chunk_body, 0)


@functools.partial(jax.jit, static_argnames=())
def kernel(level_table, type_table, feature_table, exchange_table, pair_table,
           W, b, level_indices, type_indices, feature_indices,
           exchange_indices, pair_indices):
    p1, p2, p3, p4, p5 = _project_tables(
        level_table, type_table, feature_table, exchange_table, pair_table,
        W, b)

    mesh = plsc.VectorSubcoreMesh(core_axis_name="c", subcore_axis_name="s")
    scratch = (
        [pltpu.VMEM((CHUNK,), jnp.int32) for _ in range(5)]
        + [pltpu.VMEM((CHUNK, D), jnp.float32) for _ in range(5)]
        + [pltpu.SemaphoreType.DMA]
    )
    gather_sum = functools.partial(
        pl.kernel,
        out_type=jax.ShapeDtypeStruct((N, D), jnp.float32),
        mesh=mesh,
        scratch_types=scratch,
    )(_gather_sum_body)
    return gather_sum(p1, p2, p3, p4, p5, level_indices, type_indices,
                      feature_indices, exchange_indices, pair_indices)


# speedup vs baseline: 7.5264x; 6.6825x over previous
import functools

import jax
import jax.numpy as jnp
from jax import lax
from jax.experimental import pallas as pl
from jax.experimental.pallas import tpu as pltpu
from jax.experimental.pallas import tpu_sc as plsc

N = 100000
D = 128
LANES = 16
CHUNK = 128
NFULL = N // CHUNK
TAIL = N - NFULL * CHUNK
TAIL_BASE = NFULL * CHUNK

N_LEVELS, N_TYPES, N_FEATS, N_EXCH, N_PAIRS = 50, 4, 20, 16, 1000
TFE = N_TYPES * N_FEATS * N_EXCH
FUSED = N_LEVELS * TFE
IDX_BLK = N // N_LEVELS


def _build_body(lt, tt, ft, et, li, ti, fi, ei, w1, w2, w3, w4, b,
                fused, combo):
    l = pl.program_id(0)
    dn = (((1,), (1,)), ((), ()))
    p_l = lax.dot_general(lt[...], w1[...], dn,
                          preferred_element_type=jnp.float32) + b[...]
    p_t = lax.dot_general(tt[...], w2[...], dn,
                          preferred_element_type=jnp.float32)
    p_f = lax.dot_general(ft[...], w3[...], dn,
                          preferred_element_type=jnp.float32)
    p_e = lax.dot_general(et[...], w4[...], dn,
                          preferred_element_type=jnp.float32)
    mask = lax.broadcasted_iota(jnp.int32, (N_LEVELS, 1), 0) == l
    p_l_row = jnp.sum(jnp.where(mask, p_l, 0.0), axis=0, keepdims=True)
    tfe = (p_t[:, None, None, :] + p_f[None, :, None, :]
           + p_e[None, None, :, :]).reshape(TFE, D)
    fused[...] = tfe + p_l_row
    combo[...] = ((li[...] * N_TYPES + ti[...]) * N_FEATS
                  + fi[...]) * N_EXCH + ei[...]


def _ppair_body(pt, w5, ppair):
    dn = (((1,), (1,)), ((), ()))
    ppair[...] = lax.dot_general(pt[...], w5[...], dn,
                                 preferred_element_type=jnp.float32)


def _build_tables(lt, tt, ft, et, pt, W, b, li, ti, fi, ei):
    w1 = W[:, 0:25]
    w2 = W[:, 25:50]
    w3 = W[:, 50:75]
    w4 = W[:, 75:100]
    w5 = W[:, 100:128]
    b2 = b.reshape(1, D)
    li3 = li.reshape(N_LEVELS, 1, IDX_BLK)
    ti3 = ti.reshape(N_LEVELS, 1, IDX_BLK)
    fi3 = fi.reshape(N_LEVELS, 1, IDX_BLK)
    ei3 = ei.reshape(N_LEVELS, 1, IDX_BLK)

    full = pl.BlockSpec(None, lambda l: (0, 0))
    idx_spec = pl.BlockSpec((1, 1, IDX_BLK), lambda l: (l, 0, 0))
    fused, combo = pl.pallas_call(
        _build_body,
        grid=(N_LEVELS,),
        in_specs=[
            full, full, full, full,
            idx_spec, idx_spec, idx_spec, idx_spec,
            full, full, full, full, full,
        ],
        out_specs=[
            pl.BlockSpec((TFE, D), lambda l: (l, 0)),
            idx_spec,
        ],
        out_shape=[
            jax.ShapeDtypeStruct((FUSED, D), jnp.float32),
            jax.ShapeDtypeStruct((N_LEVELS, 1, IDX_BLK), jnp.int32),
        ],
    )(lt, tt, ft, et, li3, ti3, fi3, ei3, w1, w2, w3, w4, b2)

    pair_blk = 40
    ppair = pl.pallas_call(
        _ppair_body,
        grid=(N_PAIRS // pair_blk,),
        in_specs=[pl.BlockSpec((pair_blk, 28), lambda i: (i, 0)), full],
        out_specs=pl.BlockSpec((pair_blk, D), lambda i: (i, 0)),
        out_shape=jax.ShapeDtypeStruct((N_PAIRS, D), jnp.float32),
    )(pt, w5)
    return fused, combo.reshape(N), ppair


def _gather_sum_body(fused, ppair, ci, pi, out,
                     ia0, ib0, ia1, ib1, ba0, bb0, ba1, bb1,
                     ta, tb, tba, tbb, isem, gs0, gs1, os0, os1):
    info = plsc.get_sparse_core_info()
    nc = info.num_cores
    nw = nc * info.num_subcores
    wid = lax.axis_index("s") * nc + lax.axis_index("c")
    n = (NFULL - wid + nw - 1) // nw
    max_j = (NFULL + nw - 1) // nw // 2 + 1

    isets = ((ia0, ib0), (ia1, ib1))
    bsets = ((ba0, bb0), (ba1, bb1))
    gsems = (gs0, gs1)
    osems = (os0, os1)

    def start(i, s):
        base = (wid + i * nw) * CHUNK
        ia, ib = isets[s]
        ca = pltpu.async_copy(ci.at[pl.ds(base, CHUNK)], ia, isem)
        cb = pltpu.async_copy(pi.at[pl.ds(base, CHUNK)], ib, isem)
        ca.wait()
        cb.wait()
        pltpu.async_copy(fused.at[ia], bsets[s][0], gsems[s])
        pltpu.async_copy(ppair.at[ib], bsets[s][1], gsems[s])

    def wait_gathers(s):
        ba, bb = bsets[s]
        pltpu.make_async_copy(fused.at[pl.ds(0, CHUNK)], ba, gsems[s]).wait()
        pltpu.make_async_copy(ppair.at[pl.ds(0, CHUNK)], bb, gsems[s]).wait()

    def drain_out(s):
        pltpu.make_async_copy(bsets[s][0], out.at[pl.ds(0, CHUNK)],
                              osems[s]).wait()

    def compute_store(i, s):
        base = (wid + i * nw) * CHUNK
        ba, bb = bsets[s]

        def row_body(r, _):
            for g in range(D // LANES):
                col = g * LANES
                ba[r, pl.ds(col, LANES)] = (ba[r, pl.ds(col, LANES)]
                                            + bb[r, pl.ds(col, LANES)])
            return 0

        lax.fori_loop(0, CHUNK, row_body, 0)
        pltpu.async_copy(ba, out.at[pl.ds(base, CHUNK)], osems[s])

    @pl.when(n > 0)
    def _():
        start(0, 0)

    def loop_body(j, _):
        for p in range(2):
            i = 2 * j + p

            @pl.when(i < n)
            def _():
                wait_gathers(p)

                @pl.when(i + 1 < n)
                def _():
                    @pl.when(i + 1 >= 2)
                    def _():
                        drain_out(1 - p)
                    start(i + 1, 1 - p)

                compute_store(i, p)
        return 0

    lax.fori_loop(0, max_j, loop_body, 0)

    drain_out(0)
    drain_out(1)

    @pl.when(wid == 0)
    def _():
        ca = pltpu.async_copy(ci.at[pl.ds(TAIL_BASE, TAIL)], ta, isem)
        cb = pltpu.async_copy(pi.at[pl.ds(TAIL_BASE, TAIL)], tb, isem)
        ca.wait()
        cb.wait()
        ga = pltpu.async_copy(fused.at[ta], tba, gs0)
        gb = pltpu.async_copy(ppair.at[tb], tbb, gs0)
        ga.wait()
        gb.wait()

        def row_body(r, _):
            for g in range(D // LANES):
                col = g * LANES
                tba[r, pl.ds(col, LANES)] = (tba[r, pl.ds(col, LANES)]
                                             + tbb[r, pl.ds(col, LANES)])
            return 0

        lax.fori_loop(0, TAIL, row_body, 0)
        pltpu.sync_copy(tba, out.at[pl.ds(TAIL_BASE, TAIL)])


def kernel(level_table, type_table, feature_table, exchange_table, pair_table,
           W, b, level_indices, type_indices, feature_indices,
           exchange_indices, pair_indices):
    fused, combo, ppair = _build_tables(
        level_table, type_table, feature_table, exchange_table, pair_table,
        W, b, level_indices, type_indices, feature_indices, exchange_indices)

    mesh = plsc.VectorSubcoreMesh(core_axis_name="c", subcore_axis_name="s")
    scratch = (
        [pltpu.VMEM((CHUNK,), jnp.int32) for _ in range(4)]
        + [pltpu.VMEM((CHUNK, D), jnp.float32) for _ in range(4)]
        + [pltpu.VMEM((TAIL,), jnp.int32) for _ in range(2)]
        + [pltpu.VMEM((TAIL, D), jnp.float32) for _ in range(2)]
        + [pltpu.SemaphoreType.DMA] * 5
    )
    gather_sum = functools.partial(
        pl.kernel,
        out_type=jax.ShapeDtypeStruct((N, D), jnp.float32),
        mesh=mesh,
        scratch_types=scratch,
    )(_gather_sum_body)
    return gather_sum(fused, ppair, combo, pair_indices)


# speedup vs baseline: 9.6165x; 1.2777x over previous
import functools

import jax
import jax.numpy as jnp
from jax import lax
from jax.experimental import pallas as pl
from jax.experimental.pallas import tpu as pltpu
from jax.experimental.pallas import tpu_sc as plsc

N = 100000
D = 128
LANES = 16
CHUNK = 128
NFULL = N // CHUNK
TAIL = N - NFULL * CHUNK
TAIL_BASE = NFULL * CHUNK

N_LEVELS, N_TYPES, N_FEATS, N_EXCH, N_PAIRS = 50, 4, 20, 16, 1000
TFE = N_TYPES * N_FEATS * N_EXCH
FUSED = N_LEVELS * TFE
IDX_BLK = N // N_LEVELS


G = 5
LVL_BLK = N_LEVELS // G
IDXG = N // G
PAIRG = N_PAIRS // G


def _build_body(lt, tt, ft, et, pt, li, ti, fi, ei, w1, w2, w3, w4, w5, b,
                fused, combo, ppair):
    g = pl.program_id(0)
    dn = (((1,), (1,)), ((), ()))
    p_l = lax.dot_general(lt[...], w1[...], dn,
                          preferred_element_type=jnp.float32, precision=lax.Precision.HIGHEST) + b[...]
    p_t = lax.dot_general(tt[...], w2[...], dn,
                          preferred_element_type=jnp.float32, precision=lax.Precision.HIGHEST)
    p_f = lax.dot_general(ft[...], w3[...], dn,
                          preferred_element_type=jnp.float32, precision=lax.Precision.HIGHEST)
    p_e = lax.dot_general(et[...], w4[...], dn,
                          preferred_element_type=jnp.float32, precision=lax.Precision.HIGHEST)
    ii = lax.broadcasted_iota(jnp.int32, (LVL_BLK, N_LEVELS), 0)
    jj = lax.broadcasted_iota(jnp.int32, (LVL_BLK, N_LEVELS), 1)
    sel = (jj == ii + LVL_BLK * g).astype(jnp.float32)
    p_rows = lax.dot_general(sel, p_l, (((1,), (0,)), ((), ())),
                             preferred_element_type=jnp.float32, precision=lax.Precision.HIGHEST)
    tfe = (p_t[:, None, None, :] + p_f[None, :, None, :]
           + p_e[None, None, :, :]).reshape(TFE, D)
    fused[...] = tfe[None, :, :] + p_rows[:, None, :]
    combo[...] = ((li[...] * N_TYPES + ti[...]) * N_FEATS
                  + fi[...]) * N_EXCH + ei[...]
    ppair[...] = lax.dot_general(pt[0], w5[...], dn,
                                 preferred_element_type=jnp.float32, precision=lax.Precision.HIGHEST)[None]


def _build_tables(lt, tt, ft, et, pt, W, b, li, ti, fi, ei):
    w1 = W[:, 0:25]
    w2 = W[:, 25:50]
    w3 = W[:, 50:75]
    w4 = W[:, 75:100]
    w5 = W[:, 100:128]
    b2 = b.reshape(1, D)
    li3 = li.reshape(G, 1, IDXG)
    ti3 = ti.reshape(G, 1, IDXG)
    fi3 = fi.reshape(G, 1, IDXG)
    ei3 = ei.reshape(G, 1, IDXG)
    pt3 = pt.reshape(G, PAIRG, 28)

    full = pl.BlockSpec(None, lambda g: (0, 0))
    idx_spec = pl.BlockSpec((1, 1, IDXG), lambda g: (g, 0, 0))
    fused, combo, ppair = pl.pallas_call(
        _build_body,
        grid=(G,),
        in_specs=[
            full, full, full, full,
            pl.BlockSpec((1, PAIRG, 28), lambda g: (g, 0, 0)),
            idx_spec, idx_spec, idx_spec, idx_spec,
            full, full, full, full, full, full,
        ],
        out_specs=[
            pl.BlockSpec((LVL_BLK, TFE, D), lambda g: (g, 0, 0)),
            idx_spec,
            pl.BlockSpec((1, PAIRG, D), lambda g: (g, 0, 0)),
        ],
        out_shape=[
            jax.ShapeDtypeStruct((N_LEVELS, TFE, D), jnp.float32),
            jax.ShapeDtypeStruct((G, 1, IDXG), jnp.int32),
            jax.ShapeDtypeStruct((G, PAIRG, D), jnp.float32),
        ],
    )(lt, tt, ft, et, pt3, li3, ti3, fi3, ei3, w1, w2, w3, w4, w5, b2)
    return (fused.reshape(FUSED, D), combo.reshape(N),
            ppair.reshape(N_PAIRS, D))


def _gather_sum_body(fused, ppair, ci, pi, out,
                     ia0, ib0, ia1, ib1, ba0, bb0, ba1, bb1,
                     ta, tb, tba, tbb, isem, gs0, gs1, os0, os1):
    info = plsc.get_sparse_core_info()
    nc = info.num_cores
    nw = nc * info.num_subcores
    wid = lax.axis_index("s") * nc + lax.axis_index("c")
    n = (NFULL - wid + nw - 1) // nw
    max_j = (NFULL + nw - 1) // nw // 2 + 1

    isets = ((ia0, ib0), (ia1, ib1))
    bsets = ((ba0, bb0), (ba1, bb1))
    gsems = (gs0, gs1)
    osems = (os0, os1)

    def start(i, s):
        base = (wid + i * nw) * CHUNK
        ia, ib = isets[s]
        ca = pltpu.async_copy(ci.at[pl.ds(base, CHUNK)], ia, isem)
        cb = pltpu.async_copy(pi.at[pl.ds(base, CHUNK)], ib, isem)
        ca.wait()
        cb.wait()
        pltpu.async_copy(fused.at[ia], bsets[s][0], gsems[s])
        pltpu.async_copy(ppair.at[ib], bsets[s][1], gsems[s])

    def wait_gathers(s):
        ba, bb = bsets[s]
        pltpu.make_async_copy(fused.at[pl.ds(0, CHUNK)], ba, gsems[s]).wait()
        pltpu.make_async_copy(ppair.at[pl.ds(0, CHUNK)], bb, gsems[s]).wait()

    def drain_out(s):
        pltpu.make_async_copy(bsets[s][0], out.at[pl.ds(0, CHUNK)],
                              osems[s]).wait()

    def compute_store(i, s):
        base = (wid + i * nw) * CHUNK
        ba, bb = bsets[s]

        def row_body(r, _):
            for g in range(D // LANES):
                col = g * LANES
                ba[r, pl.ds(col, LANES)] = (ba[r, pl.ds(col, LANES)]
                                            + bb[r, pl.ds(col, LANES)])
            return 0

        lax.fori_loop(0, CHUNK, row_body, 0)
        pltpu.async_copy(ba, out.at[pl.ds(base, CHUNK)], osems[s])

    @pl.when(n > 0)
    def _():
        start(0, 0)

    def loop_body(j, _):
        for p in range(2):
            i = 2 * j + p

            @pl.when(i < n)
            def _():
                wait_gathers(p)

                @pl.when(i + 1 < n)
                def _():
                    @pl.when(i + 1 >= 2)
                    def _():
                        drain_out(1 - p)
                    start(i + 1, 1 - p)

                compute_store(i, p)
        return 0

    lax.fori_loop(0, max_j, loop_body, 0)

    drain_out(0)
    drain_out(1)

    @pl.when(wid == 0)
    def _():
        ca = pltpu.async_copy(ci.at[pl.ds(TAIL_BASE, TAIL)], ta, isem)
        cb = pltpu.async_copy(pi.at[pl.ds(TAIL_BASE, TAIL)], tb, isem)
        ca.wait()
        cb.wait()
        ga = pltpu.async_copy(fused.at[ta], tba, gs0)
        gb = pltpu.async_copy(ppair.at[tb], tbb, gs0)
        ga.wait()
        gb.wait()

        def row_body(r, _):
            for g in range(D // LANES):
                col = g * LANES
                tba[r, pl.ds(col, LANES)] = (tba[r, pl.ds(col, LANES)]
                                             + tbb[r, pl.ds(col, LANES)])
            return 0

        lax.fori_loop(0, TAIL, row_body, 0)
        pltpu.sync_copy(tba, out.at[pl.ds(TAIL_BASE, TAIL)])


def kernel(level_table, type_table, feature_table, exchange_table, pair_table,
           W, b, level_indices, type_indices, feature_indices,
           exchange_indices, pair_indices):
    fused, combo, ppair = _build_tables(
        level_table, type_table, feature_table, exchange_table, pair_table,
        W, b, level_indices, type_indices, feature_indices, exchange_indices)

    mesh = plsc.VectorSubcoreMesh(core_axis_name="c", subcore_axis_name="s")
    scratch = (
        [pltpu.VMEM((CHUNK,), jnp.int32) for _ in range(4)]
        + [pltpu.VMEM((CHUNK, D), jnp.float32) for _ in range(4)]
        + [pltpu.VMEM((TAIL,), jnp.int32) for _ in range(2)]
        + [pltpu.VMEM((TAIL, D), jnp.float32) for _ in range(2)]
        + [pltpu.SemaphoreType.DMA] * 5
    )
    gather_sum = functools.partial(
        pl.kernel,
        out_type=jax.ShapeDtypeStruct((N, D), jnp.float32),
        mesh=mesh,
        scratch_types=scratch,
    )(_gather_sum_body)
    return gather_sum(fused, ppair, combo, pair_indices)
